# baseline (speedup 1.0000x reference)
import numpy as np

import jax
import jax.numpy as jnp
from jax.experimental import pallas as pl
from jax.experimental.pallas import tpu as pltpu

_KTOP = 50
_BF = jnp.bfloat16


def _f32_key(v):
    b = int(np.float32(v).view(np.int32))
    return b ^ 0x7FFFFFFF if b < 0 else b


_LO_KEY = _f32_key(-1.02)
_KEY_ITERS = 31


def _dot(a, b):
    return jnp.dot(a, b, preferred_element_type=jnp.float32)


def _dinv_of(d):
    return jnp.where(d > 0, 1.0 / jnp.sqrt(jnp.maximum(d, 1e-12)), 0.0)


def _degscale_body(adj_ref, x_ref, w_ref, dinv_ref, m_ref):
    d = jnp.sum(adj_ref[...], axis=1, keepdims=True)
    dinv_ref[...] = _dinv_of(d)
    m_ref[...] = (x_ref[...] * w_ref[...]).astype(_BF)


def _enc1_body(adj_ref, m_ref, dinv_blk_ref, dinvT_ref, w2_ref, m2_ref):
    anb = ((dinv_blk_ref[...] * adj_ref[...]) * dinvT_ref[...]).astype(_BF)
    h = jnp.tanh(_dot(anb, m_ref[...]))
    m2_ref[...] = (h * w2_ref[...]).astype(_BF)


def _enc2_body(adj_ref, m_ref, dinv_blk_ref, dinvT_ref, e_ref):
    anb = ((dinv_blk_ref[...] * adj_ref[...]) * dinvT_ref[...]).astype(_BF)
    ep = _dot(anb, m_ref[...])
    nrm = jnp.sqrt(jnp.sum(ep * ep, axis=1, keepdims=True))
    e_ref[...] = (ep / (nrm + 1e-12)).astype(_BF)


def _thr_body(eb_ref, eT_ref, t_ref):
    sim = _dot(eb_ref[...], eT_ref[...])
    r, n = sim.shape
    i32 = jnp.int32
    flip = i32(0x7FFFFFFF)
    b = jax.lax.bitcast_convert_type(sim, i32)
    keys = jnp.where(b < 0, b ^ flip, b)

    csz = 64
    nchunk = (n + csz - 1) // csz
    pad = nchunk * csz - n
    kp = keys
    if pad:
        kp = jnp.concatenate(
            [keys, jnp.full((r, pad), i32(-0x7F000000), i32)], axis=1)
    cm = jnp.concatenate(
        [jnp.max(kp[:, q * csz:(q + 1) * csz], axis=1, keepdims=True)
         for q in range(nchunk)], axis=1)
    kmax = jnp.max(cm, axis=1, keepdims=True)

    def cbody(_, lohi):
        lo, hi = lohi
        mid = lo + jax.lax.shift_right_arithmetic(hi - lo, 1)
        cnt = jnp.sum((cm >= mid).astype(i32), axis=1, keepdims=True)
        ge = cnt >= _KTOP
        return (jnp.where(ge, mid, lo), jnp.where(ge, hi, mid))

    cm50, _ = jax.lax.fori_loop(
        0, _KEY_ITERS, cbody,
        (jnp.full((r, 1), _LO_KEY, i32), kmax + 1))

    def cond(lohi):
        lo, hi = lohi
        return jnp.any(hi - lo > 1)

    def body(lohi):
        lo, hi = lohi
        mid = lo + jax.lax.shift_right_arithmetic(hi - lo, 1)
        cnt = jnp.sum((keys >= mid).astype(i32), axis=1, keepdims=True)
        ge = cnt >= _KTOP
        return (jnp.where(ge, mid, lo), jnp.where(ge, hi, mid))

    lo, hi = jax.lax.while_loop(cond, body, (cm50, kmax + 1))
    tk = jnp.where(lo < 0, lo ^ flip, lo)
    t_ref[...] = jax.lax.bitcast_convert_type(tk, jnp.float32)


def _asm_body(eb_ref, eT_ref, tb_ref, tT_ref, adj_ref, x_ref, w1_ref,
              anew_ref, afin_ref, dfi_ref, xw_ref):
    sim = _dot(eb_ref[...], eT_ref[...])
    keep = (sim >= tb_ref[...]).astype(jnp.float32) \
        + (sim >= tT_ref[...]).astype(jnp.float32)
    anew = 0.5 * sim * keep
    anew_ref[...] = anew
    afin = anew + adj_ref[...]
    afin_ref[...] = afin
    dfi_ref[...] = _dinv_of(jnp.sum(afin, axis=1, keepdims=True))
    xw_ref[...] = _dot(x_ref[...].astype(_BF), w1_ref[...].astype(_BF)) \
        .astype(_BF)


def _gcn1_body(afin_ref, xw_ref, dfi_blk_ref, dfiT_ref, w2_ref, zw_ref):
    afnb = ((dfi_blk_ref[...] * afin_ref[...]) * dfiT_ref[...]).astype(_BF)
    z = jnp.maximum(_dot(afnb, xw_ref[...]), 0.0)
    zw_ref[...] = _dot(z.astype(_BF), w2_ref[...].astype(_BF)).astype(_BF)


def _gcn2_body(afin_ref, zw_ref, dfi_blk_ref, dfiT_ref, out_ref):
    afnb = ((dfi_blk_ref[...] * afin_ref[...]) * dfiT_ref[...]).astype(_BF)
    out_ref[...] = _dot(afnb, zw_ref[...])


def _full(shape):
    return pl.BlockSpec(shape, lambda i: (0,) * len(shape))


def _rows(r, ncols):
    return pl.BlockSpec((r, ncols), lambda i: (i, 0))


def _shard_pipeline(x, Adj, w_g1, w_g2, W1, W2, interpret=False, axis=None):
    nl, d = x.shape
    n = Adj.shape[1]
    h_dim = W1.shape[1]
    c_dim = W2.shape[1]
    r_mm = 200 if nl % 200 == 0 else nl
    r_sel = 40 if nl % 40 == 0 else nl
    r_thr = 200 if nl % 200 == 0 else r_sel

    def ag(v):
        if axis is None:
            return v
        return jax.lax.all_gather(v, axis, axis=0, tiled=True)

    def call(body, grid, in_specs, out_specs, out_shape):
        kwargs = {}
        if not interpret:
            kwargs["compiler_params"] = pltpu.CompilerParams(
                dimension_semantics=("parallel",))
        return pl.pallas_call(
            body,
            grid=grid,
            in_specs=in_specs,
            out_specs=out_specs,
            out_shape=out_shape,
            interpret=interpret,
            **kwargs,
        )

    f32 = jnp.float32
    wg1r = w_g1.reshape(1, d)
    wg2r = w_g2.reshape(1, d)

    dinv, m1 = call(
        _degscale_body, (nl // r_mm,),
        [_rows(r_mm, n), _rows(r_mm, d), _full((1, d))],
        (_rows(r_mm, 1), _rows(r_mm, d)),
        (jax.ShapeDtypeStruct((nl, 1), f32),
         jax.ShapeDtypeStruct((nl, d), _BF)))(Adj, x, wg1r)
    dinvT = ag(dinv).reshape(1, n)
    m1f = ag(m1)

    m2 = call(_enc1_body, (nl // r_mm,),
              [_rows(r_mm, n), _full((n, d)), _rows(r_mm, 1), _full((1, n)),
               _full((1, d))],
              _rows(r_mm, d),
              jax.ShapeDtypeStruct((nl, d), _BF))(Adj, m1f, dinv, dinvT, wg2r)
    m2f = ag(m2)

    e = call(_enc2_body, (nl // r_mm,),
             [_rows(r_mm, n), _full((n, d)), _rows(r_mm, 1), _full((1, n))],
             _rows(r_mm, d),
             jax.ShapeDtypeStruct((nl, d), _BF))(Adj, m2f, dinv, dinvT)
    eT = ag(e).T

    t = call(_thr_body, (nl // r_thr,),
             [_rows(r_thr, d), _full((d, n))],
             _rows(r_thr, 1),
             jax.ShapeDtypeStruct((nl, 1), f32))(e, eT)
    tT = ag(t).reshape(1, n)

    anew, afin, dfi, xw = call(
        _asm_body, (nl // r_sel,),
        [_rows(r_sel, d), _full((d, n)), _rows(r_sel, 1), _full((1, n)),
         _rows(r_sel, n), _rows(r_sel, d), _full((d, h_dim))],
        (_rows(r_sel, n), _rows(r_sel, n), _rows(r_sel, 1),
         _rows(r_sel, h_dim)),
        (jax.ShapeDtypeStruct((nl, n), f32),
         jax.ShapeDtypeStruct((nl, n), f32),
         jax.ShapeDtypeStruct((nl, 1), f32),
         jax.ShapeDtypeStruct((nl, h_dim), _BF)))(e, eT, t, tT, Adj, x, W1)
    dfiT = ag(dfi).reshape(1, n)
    xwf = ag(xw)

    zw = call(_gcn1_body, (nl // r_mm,),
              [_rows(r_mm, n), _full((n, h_dim)), _rows(r_mm, 1),
               _full((1, n)), _full((h_dim, c_dim))],
              _rows(r_mm, c_dim),
              jax.ShapeDtypeStruct((nl, c_dim), _BF))(afin, xwf, dfi, dfiT, W2)
    zwf = ag(zw)

    out = call(_gcn2_body, (nl // r_mm,),
               [_rows(r_mm, n), _full((n, c_dim)), _rows(r_mm, 1),
                _full((1, n))],
               _rows(r_mm, c_dim),
               jax.ShapeDtypeStruct((nl, c_dim), f32))(afin, zwf, dfi, dfiT)

    return (out, anew, afin)


def _pipeline(x, Adj, w_g1, w_g2, W1, W2, interpret=False):
    import functools
    from jax.sharding import Mesh, PartitionSpec as P
    try:
        from jax.experimental.shard_map import shard_map
    except ImportError:
        shard_map = None

    n = x.shape[0]
    devs = jax.devices()
    nd = 2 if (shard_map is not None and len(devs) >= 2
               and n % 400 == 0) else 1
    if nd == 1:
        return _shard_pipeline(x, Adj, w_g1, w_g2, W1, W2,
                               interpret=interpret, axis=None)
    mesh = Mesh(np.array(devs[:nd]), ("r",))
    body = functools.partial(_shard_pipeline, interpret=interpret, axis="r")
    fn = shard_map(body, mesh=mesh,
                   in_specs=(P("r"), P("r"), P(), P(), P(), P()),
                   out_specs=(P("r"), P("r"), P("r")),
                   check_rep=False)
    return fn(x, Adj, w_g1, w_g2, W1, W2)


def kernel(x, Adj, w_g1, w_g2, W1, W2):
    return _pipeline(x, Adj, w_g1, w_g2, W1, W2)


# speedup vs baseline: 1.0994x; 1.0994x over previous
import numpy as np

import jax
import jax.numpy as jnp
from jax.experimental import pallas as pl
from jax.experimental.pallas import tpu as pltpu

_KTOP = 50
_BF = jnp.bfloat16


def _f32_key(v):
    b = int(np.float32(v).view(np.int32))
    return b ^ 0x7FFFFFFF if b < 0 else b


_LO_KEY = _f32_key(-1.02)
_KEY_ITERS = 31


def _dot(a, b):
    return jnp.dot(a, b, preferred_element_type=jnp.float32)


def _dinv_of(d):
    return jnp.where(d > 0, 1.0 / jnp.sqrt(jnp.maximum(d, 1e-12)), 0.0)


def _degscale_body(adj_ref, x_ref, w_ref, dinv_ref, m_ref):
    d = jnp.sum(adj_ref[...], axis=1, keepdims=True)
    dinv_ref[...] = _dinv_of(d)
    m_ref[...] = (x_ref[...] * w_ref[...]).astype(_BF)


def _enc1_body(adj_ref, m_ref, dinv_blk_ref, dinvT_ref, w2_ref, m2_ref):
    anb = ((dinv_blk_ref[...] * adj_ref[...]) * dinvT_ref[...]).astype(_BF)
    h = jnp.tanh(_dot(anb, m_ref[...]))
    m2_ref[...] = (h * w2_ref[...]).astype(_BF)


def _enc2_body(adj_ref, m_ref, dinv_blk_ref, dinvT_ref, e_ref):
    anb = ((dinv_blk_ref[...] * adj_ref[...]) * dinvT_ref[...]).astype(_BF)
    ep = _dot(anb, m_ref[...])
    nrm = jnp.sqrt(jnp.sum(ep * ep, axis=1, keepdims=True))
    e_ref[...] = (ep / (nrm + 1e-12)).astype(_BF)


def _thr_body(eb_ref, eT_ref, t_ref):
    sim = _dot(eb_ref[...], eT_ref[...])
    r, n = sim.shape
    i32 = jnp.int32
    flip = i32(0x7FFFFFFF)
    b = jax.lax.bitcast_convert_type(sim, i32)
    keys = jnp.where(b < 0, b ^ flip, b)

    csz = 128
    nchunk = (n + csz - 1) // csz
    pad = nchunk * csz - n
    kp = keys
    if pad:
        kp = jnp.concatenate(
            [keys, jnp.full((r, pad), i32(-0x7F000000), i32)], axis=1)
    cm = jnp.concatenate(
        [jnp.max(kp[:, q * csz:(q + 1) * csz], axis=1, keepdims=True)
         for q in range(nchunk)], axis=1)
    kmax = jnp.max(cm, axis=1, keepdims=True)

    def cbody(_, lohi):
        lo, hi = lohi
        mid = lo + jax.lax.shift_right_arithmetic(hi - lo, 1)
        cnt = jnp.sum((cm >= mid).astype(i32), axis=1, keepdims=True)
        ge = cnt >= _KTOP
        return (jnp.where(ge, mid, lo), jnp.where(ge, hi, mid))

    cm50, _ = jax.lax.fori_loop(
        0, _KEY_ITERS, cbody,
        (jnp.full((r, 1), _LO_KEY, i32), kmax + 1))

    def cond(lohi):
        lo, hi = lohi
        return jnp.any(hi - lo > 1)

    def body(lohi):
        lo, hi = lohi
        mid = lo + jax.lax.shift_right_arithmetic(hi - lo, 1)
        cnt = jnp.sum((keys >= mid).astype(i32), axis=1, keepdims=True)
        ge = cnt >= _KTOP
        return (jnp.where(ge, mid, lo), jnp.where(ge, hi, mid))

    lo, hi = jax.lax.while_loop(cond, body, (cm50, kmax + 1))
    tk = jnp.where(lo < 0, lo ^ flip, lo)
    t_ref[...] = jax.lax.bitcast_convert_type(tk, jnp.float32)


def _asm_body(eb_ref, eT_ref, tb_ref, tT_ref, adj_ref, x_ref, w1_ref,
              anew_ref, afin_ref, dfi_ref, xw_ref):
    sim = _dot(eb_ref[...], eT_ref[...])
    keep = (sim >= tb_ref[...]).astype(jnp.float32) \
        + (sim >= tT_ref[...]).astype(jnp.float32)
    anew = 0.5 * sim * keep
    anew_ref[...] = anew
    afin = anew + adj_ref[...]
    afin_ref[...] = afin
    dfi_ref[...] = _dinv_of(jnp.sum(afin, axis=1, keepdims=True))
    xw_ref[...] = _dot(x_ref[...].astype(_BF), w1_ref[...].astype(_BF)) \
        .astype(_BF)


def _gcn1_body(afin_ref, xw_ref, dfi_blk_ref, dfiT_ref, w2_ref, zw_ref):
    afnb = ((dfi_blk_ref[...] * afin_ref[...]) * dfiT_ref[...]).astype(_BF)
    z = jnp.maximum(_dot(afnb, xw_ref[...]), 0.0)
    zw_ref[...] = _dot(z.astype(_BF), w2_ref[...].astype(_BF)).astype(_BF)


def _gcn2_body(afin_ref, zw_ref, dfi_blk_ref, dfiT_ref, out_ref):
    afnb = ((dfi_blk_ref[...] * afin_ref[...]) * dfiT_ref[...]).astype(_BF)
    out_ref[...] = _dot(afnb, zw_ref[...])


def _full(shape):
    return pl.BlockSpec(shape, lambda i: (0,) * len(shape))


def _rows(r, ncols):
    return pl.BlockSpec((r, ncols), lambda i: (i, 0))


def _shard_pipeline(x, Adj, w_g1, w_g2, W1, W2, interpret=False, axis=None):
    nl, d = x.shape
    n = Adj.shape[1]
    h_dim = W1.shape[1]
    c_dim = W2.shape[1]
    r_mm = 200 if nl % 200 == 0 else nl
    r_sel = 40 if nl % 40 == 0 else nl
    r_thr = 200 if nl % 200 == 0 else r_sel

    def ag(v):
        if axis is None:
            return v
        return jax.lax.all_gather(v, axis, axis=0, tiled=True)

    def call(body, grid, in_specs, out_specs, out_shape):
        kwargs = {}
        if not interpret:
            kwargs["compiler_params"] = pltpu.CompilerParams(
                dimension_semantics=("parallel",))
        return pl.pallas_call(
            body,
            grid=grid,
            in_specs=in_specs,
            out_specs=out_specs,
            out_shape=out_shape,
            interpret=interpret,
            **kwargs,
        )

    f32 = jnp.float32
    wg1r = w_g1.reshape(1, d)
    wg2r = w_g2.reshape(1, d)

    dinv, m1 = call(
        _degscale_body, (nl // r_mm,),
        [_rows(r_mm, n), _rows(r_mm, d), _full((1, d))],
        (_rows(r_mm, 1), _rows(r_mm, d)),
        (jax.ShapeDtypeStruct((nl, 1), f32),
         jax.ShapeDtypeStruct((nl, d), _BF)))(Adj, x, wg1r)
    dinvT = ag(dinv).reshape(1, n)
    m1f = ag(m1)

    m2 = call(_enc1_body, (nl // r_mm,),
              [_rows(r_mm, n), _full((n, d)), _rows(r_mm, 1), _full((1, n)),
               _full((1, d))],
              _rows(r_mm, d),
              jax.ShapeDtypeStruct((nl, d), _BF))(Adj, m1f, dinv, dinvT, wg2r)
    m2f = ag(m2)

    e = call(_enc2_body, (nl // r_mm,),
             [_rows(r_mm, n), _full((n, d)), _rows(r_mm, 1), _full((1, n))],
             _rows(r_mm, d),
             jax.ShapeDtypeStruct((nl, d), _BF))(Adj, m2f, dinv, dinvT)
    eT = ag(e).T

    t = call(_thr_body, (nl // r_thr,),
             [_rows(r_thr, d), _full((d, n))],
             _rows(r_thr, 1),
             jax.ShapeDtypeStruct((nl, 1), f32))(e, eT)
    tT = ag(t).reshape(1, n)

    anew, afin, dfi, xw = call(
        _asm_body, (nl // r_sel,),
        [_rows(r_sel, d), _full((d, n)), _rows(r_sel, 1), _full((1, n)),
         _rows(r_sel, n), _rows(r_sel, d), _full((d, h_dim))],
        (_rows(r_sel, n), _rows(r_sel, n), _rows(r_sel, 1),
         _rows(r_sel, h_dim)),
        (jax.ShapeDtypeStruct((nl, n), f32),
         jax.ShapeDtypeStruct((nl, n), f32),
         jax.ShapeDtypeStruct((nl, 1), f32),
         jax.ShapeDtypeStruct((nl, h_dim), _BF)))(e, eT, t, tT, Adj, x, W1)
    dfiT = ag(dfi).reshape(1, n)
    xwf = ag(xw)

    zw = call(_gcn1_body, (nl // r_mm,),
              [_rows(r_mm, n), _full((n, h_dim)), _rows(r_mm, 1),
               _full((1, n)), _full((h_dim, c_dim))],
              _rows(r_mm, c_dim),
              jax.ShapeDtypeStruct((nl, c_dim), _BF))(afin, xwf, dfi, dfiT, W2)
    zwf = ag(zw)

    out = call(_gcn2_body, (nl // r_mm,),
               [_rows(r_mm, n), _full((n, c_dim)), _rows(r_mm, 1),
                _full((1, n))],
               _rows(r_mm, c_dim),
               jax.ShapeDtypeStruct((nl, c_dim), f32))(afin, zwf, dfi, dfiT)

    return (out, anew, afin)


def _pipeline(x, Adj, w_g1, w_g2, W1, W2, interpret=False):
    import functools
    from jax.sharding import Mesh, PartitionSpec as P
    try:
        from jax.experimental.shard_map import shard_map
    except ImportError:
        shard_map = None

    n = x.shape[0]
    devs = jax.devices()
    nd = 2 if (shard_map is not None and len(devs) >= 2
               and n % 400 == 0) else 1
    if nd == 1:
        return _shard_pipeline(x, Adj, w_g1, w_g2, W1, W2,
                               interpret=interpret, axis=None)
    mesh = Mesh(np.array(devs[:nd]), ("r",))
    body = functools.partial(_shard_pipeline, interpret=interpret, axis="r")
    fn = shard_map(body, mesh=mesh,
                   in_specs=(P("r"), P("r"), P(), P(), P(), P()),
                   out_specs=(P("r"), P("r"), P("r")),
                   check_rep=False)
    return fn(x, Adj, w_g1, w_g2, W1, W2)


def kernel(x, Adj, w_g1, w_g2, W1, W2):
    return _pipeline(x, Adj, w_g1, w_g2, W1, W2)
